# Initial kernel scaffold; baseline (speedup 1.0000x reference)
import jax
import jax.numpy as jnp
from jax.experimental import pallas as pl


def kernel(updates, indices):
    raise NotImplementedError("write your pallas kernel here")


# speedup vs baseline: 8.2830x; 8.2830x over previous
import jax
import jax.numpy as jnp
from jax import lax
from jax.experimental import pallas as pl
from jax.experimental.pallas import tpu as pltpu
from jax.experimental.pallas import tpu_sc as plsc

B, H, W, C = 4, 112, 112, 96
HW = H * W
OHW = 4 * HW
CW = 16
NQ = C // CW
NWIN = B * NQ
NSUB = 16
WIN_PER_CORE = NWIN // 2
RPS = HW // NSUB
EPS = RPS * CW
CHUNK = 128
NCH = EPS // CHUNK
ACC = OHW * CW
ZSPAN = ACC // NSUB
ZB = 6272


def _sc_body(upd_hbm, idx_hbm, out_hbm, acc, vstage, rawi, dsts, vals, zbuf):
    cid = lax.axis_index("c")
    sid = lax.axis_index("s")
    lanes = lax.iota(jnp.int32, 16)

    @pl.loop(0, ZB // 16)
    def _(i):
        zbuf[pl.ds(i * 16, 16)] = jnp.zeros((16,), jnp.float32)

    @pl.loop(0, WIN_PER_CORE)
    def _(t):
        w = cid * WIN_PER_CORE + t
        b = w // NQ
        c0 = (w % NQ) * CW
        z0 = sid * ZSPAN

        @pl.loop(0, ZSPAN // ZB)
        def _(z):
            pltpu.sync_copy(zbuf, acc.at[pl.ds(z0 + z * ZB, ZB)])
        plsc.subcore_barrier()

        r0 = sid * RPS
        pltpu.sync_copy(idx_hbm.at[b, pl.ds(r0, RPS), pl.ds(c0, CW)], rawi)
        pltpu.sync_copy(upd_hbm.at[b, pl.ds(r0, RPS), pl.ds(c0, CW)], vstage)

        @pl.loop(0, RPS)
        def _(r):
            rv = rawi[r, pl.ds(0, CW)]
            y = lax.shift_right_logical(rv, 5).astype(jnp.float32)
            p = (y * jnp.float32(1.0 / 3.0)).astype(jnp.int32)
            dsts[pl.ds(r * CW, CW)] = p * CW + lanes
            vals[pl.ds(r * CW, CW)] = vstage[r, pl.ds(0, CW)]

        @pl.loop(0, NCH)
        def _(ci):
            pltpu.sync_copy(vals.at[pl.ds(ci * CHUNK, CHUNK)],
                            acc.at[dsts.at[pl.ds(ci * CHUNK, CHUNK)]],
                            add=True)
        plsc.subcore_barrier()

        pltpu.sync_copy(acc.at[pl.ds(z0, ZSPAN)],
                        out_hbm.at[w, pl.ds(z0, ZSPAN)])
        plsc.subcore_barrier()


_sc_call = pl.kernel(
    _sc_body,
    out_type=jax.ShapeDtypeStruct((NWIN, ACC), jnp.float32),
    mesh=plsc.VectorSubcoreMesh(core_axis_name="c", subcore_axis_name="s"),
    scratch_types=[
        pltpu.VMEM_SHARED((ACC,), jnp.float32),
        pltpu.VMEM((RPS, CW), jnp.float32),
        pltpu.VMEM((RPS, CW), jnp.int32),
        pltpu.VMEM((EPS,), jnp.int32),
        pltpu.VMEM((EPS,), jnp.float32),
        pltpu.VMEM((ZB,), jnp.float32),
    ],
    compiler_params=pltpu.CompilerParams(
        use_tc_tiling_on_sc=False, needs_layout_passes=False),
)

TP = 1024


def _relayout_body(*refs):
    out_ref = refs[-1]
    for q in range(NQ):
        out_ref[0, :, q, :] = refs[q][0]


_relayout = pl.pallas_call(
    _relayout_body,
    grid=(B, OHW // TP),
    in_specs=[
        pl.BlockSpec((1, TP, CW), lambda b, i, q=q: (b * NQ + q, i, 0))
        for q in range(NQ)
    ],
    out_specs=pl.BlockSpec((1, TP, NQ, CW), lambda b, i: (b, i, 0, 0)),
    out_shape=jax.ShapeDtypeStruct((B, OHW, NQ, CW), jnp.float32),
)


@jax.jit
def kernel(updates, indices):
    upd = updates.reshape(B, HW, C)
    idx = indices.astype(jnp.int32).reshape(B, HW, C)
    win = _sc_call(upd, idx)
    win3 = win.reshape(NWIN, OHW, CW)
    out = _relayout(*([win3] * NQ))
    return out.reshape(B, 2 * H, 2 * W, C)


# speedup vs baseline: 13.3505x; 1.6118x over previous
import jax
import jax.numpy as jnp
from jax import lax
from jax.experimental import pallas as pl
from jax.experimental.pallas import tpu as pltpu
from jax.experimental.pallas import tpu_sc as plsc

B, H, W, C = 4, 112, 112, 96
HW = H * W
OHW = 4 * HW
CW = 16
NQ = C // CW
NWIN = B * NQ
NSUB = 16
WIN_PER_CORE = NWIN // 2
RPS = HW // NSUB
EPS = RPS * CW
CHUNK = 128
NCH = EPS // CHUNK
ACC = OHW * CW
ZSPAN = ACC // NSUB
ZB = 6272


def _sc_body(upd_hbm, idx_hbm, out_hbm, acc, vstage, rawi, dsts, vals, zbuf):
    cid = lax.axis_index("c")
    sid = lax.axis_index("s")
    lanes = lax.iota(jnp.int32, 16)

    @pl.loop(0, ZB // 16)
    def _(i):
        zbuf[pl.ds(i * 16, 16)] = jnp.zeros((16,), jnp.float32)

    @pl.loop(0, WIN_PER_CORE)
    def _(t):
        w = cid * WIN_PER_CORE + t
        b = w // NQ
        c0 = (w % NQ) * CW
        z0 = sid * ZSPAN

        @pl.loop(0, ZSPAN // ZB)
        def _(z):
            pltpu.sync_copy(zbuf, acc.at[pl.ds(z0 + z * ZB, ZB)])
        plsc.subcore_barrier()

        r0 = sid * RPS
        pltpu.sync_copy(idx_hbm.at[b, pl.ds(r0, RPS), pl.ds(c0, CW)], rawi)
        pltpu.sync_copy(upd_hbm.at[b, pl.ds(r0, RPS), pl.ds(c0, CW)], vstage)

        @pl.loop(0, RPS)
        def _(r):
            rv = rawi[r, pl.ds(0, CW)]
            y = lax.shift_right_logical(rv, 5).astype(jnp.float32)
            p = (y * jnp.float32(1.0 / 3.0)).astype(jnp.int32)
            dsts[pl.ds(r * CW, CW)] = p * CW + lanes
            vals[pl.ds(r * CW, CW)] = vstage[r, pl.ds(0, CW)]

        @pl.loop(0, NCH)
        def _(ci):
            pltpu.sync_copy(vals.at[pl.ds(ci * CHUNK, CHUNK)],
                            acc.at[dsts.at[pl.ds(ci * CHUNK, CHUNK)]],
                            add=True)
        plsc.subcore_barrier()

        pltpu.sync_copy(acc.at[pl.ds(z0, ZSPAN)],
                        out_hbm.at[w, pl.ds(z0, ZSPAN)])
        plsc.subcore_barrier()


_sc_call = pl.kernel(
    _sc_body,
    out_type=jax.ShapeDtypeStruct((NWIN, ACC), jnp.float32),
    mesh=plsc.VectorSubcoreMesh(core_axis_name="c", subcore_axis_name="s"),
    scratch_types=[
        pltpu.VMEM_SHARED((ACC,), jnp.float32),
        pltpu.VMEM((RPS, CW), jnp.float32),
        pltpu.VMEM((RPS, CW), jnp.int32),
        pltpu.VMEM((EPS,), jnp.int32),
        pltpu.VMEM((EPS,), jnp.float32),
        pltpu.VMEM((ZB,), jnp.float32),
    ],
    compiler_params=pltpu.CompilerParams(
        use_tc_tiling_on_sc=False, needs_layout_passes=False),
)

TP = 1024


def _relayout_body(*refs):
    out_ref = refs[-1]
    for q in range(NQ):
        out_ref[0, :, q * CW:(q + 1) * CW] = refs[q][0]


_relayout = pl.pallas_call(
    _relayout_body,
    grid=(B, OHW // TP),
    in_specs=[
        pl.BlockSpec((1, TP, CW), lambda b, i, q=q: (b * NQ + q, i, 0))
        for q in range(NQ)
    ],
    out_specs=pl.BlockSpec((1, TP, C), lambda b, i: (b, i, 0)),
    out_shape=jax.ShapeDtypeStruct((B, OHW, C), jnp.float32),
)


@jax.jit
def kernel(updates, indices):
    upd = updates.reshape(B, HW, C)
    idx = indices.astype(jnp.int32).reshape(B, HW, C)
    win = _sc_call(upd, idx)
    win3 = win.reshape(NWIN, OHW, CW)
    out = _relayout(*([win3] * NQ))
    return out.reshape(B, 2 * H, 2 * W, C)
